# baseline (speedup 1.0000x reference)
import functools

import jax
import jax.numpy as jnp
from jax import lax
from jax.experimental import pallas as pl
from jax.experimental.pallas import tpu as pltpu
from jax.experimental.pallas import tpu_sc as plsc

N = 10000
E = 320000
D = 128

NC = 2
NS = 16
NPAD = 10240
CH = 80
EDGES_PER_TILE = E // (NC * NS)
NCHUNK = EDGES_PER_TILE // CH
ROWS_PER_TILE = NPAD // NS
ZROWS = 128
NSTRIPE = ROWS_PER_TILE // ZROWS

NBUF = 4
NIDX = 8
AH_G = 3
AH_F = 6
GRP = 8
NGRP = (NCHUNK - GRP + AH_G) // GRP
EPI0 = NGRP * GRP


def _seg_sum_partials(h, edge_index):
    mesh = plsc.VectorSubcoreMesh(core_axis_name="c", subcore_axis_name="s")

    @functools.partial(
        pl.kernel,
        mesh=mesh,
        out_type=jax.ShapeDtypeStruct((NC, NPAD, D), jnp.float32),
        scratch_types=(
            [pltpu.VMEM((CH, D), jnp.float32) for _ in range(NBUF)]
            + [pltpu.VMEM((2, CH), jnp.int32) for _ in range(NIDX)]
            + [pltpu.VMEM_SHARED((NPAD, D), jnp.float32)]
            + [pltpu.SemaphoreType.DMA] * (2 * NBUF + NIDX)
        ),
    )
    def k(h_hbm, ei_hbm, out_hbm, *refs):
        rows = refs[0:NBUF]
        idx = refs[NBUF:NBUF + NIDX]
        acc = refs[NBUF + NIDX]
        sems = refs[NBUF + NIDX + 1:]
        gsem = sems[0:NBUF]
        ssem = sems[NBUF:2 * NBUF]
        isem = sems[2 * NBUF:]

        c = lax.axis_index("c")
        s = lax.axis_index("s")
        base = c * (E // NC) + s * EDGES_PER_TILE

        def fetch_idx(j, q):
            off = base + j * CH
            pltpu.async_copy(ei_hbm.at[pl.ds(off, CH)], idx[q].at[0], isem[q])
            pltpu.async_copy(
                ei_hbm.at[pl.ds(E + off, CH)], idx[q].at[1], isem[q])

        def wait_idx(j, q):
            off = base + j * CH
            pltpu.make_async_copy(
                ei_hbm.at[pl.ds(off, CH)], idx[q].at[0], isem[q]).wait()
            pltpu.make_async_copy(
                ei_hbm.at[pl.ds(E + off, CH)], idx[q].at[1], isem[q]).wait()

        for j in range(AH_F):
            fetch_idx(j, j)

        zb = NBUF - 1

        def zfill(t, carry):
            i = t // (D // 16)
            j = t % (D // 16)
            rows[zb][i, pl.ds(j * 16, 16)] = jnp.zeros((16,), jnp.float32)
            return carry

        lax.fori_loop(0, CH * (D // 16), zfill, 0)

        row0 = s * ROWS_PER_TILE
        for t in range(ROWS_PER_TILE // CH):
            pltpu.async_copy(rows[zb], acc.at[pl.ds(row0 + t * CH, CH)],
                             ssem[zb])
        for j in range(AH_G):
            wait_idx(j, j)
            pltpu.async_copy(h_hbm.at[pl.ds((j * CH) % 9920, CH)],
                             rows[j], gsem[j])
        for t in range(ROWS_PER_TILE // CH):
            pltpu.make_async_copy(
                rows[zb], acc.at[pl.ds(row0 + t * CH, CH)], ssem[zb]).wait()
        plsc.subcore_barrier()

        def emit_steady(g, b):
            j = g * GRP + b
            static = isinstance(j, int)
            jprev = NBUF - AH_G
            rb = b % NBUF
            rbg = (b + AH_G) % NBUF
            q = b % NIDX
            qg = (b + AH_G) % NIDX
            qf = (b + AH_F) % NIDX
            qp = (b + NIDX - jprev) % NIDX

            def fire_fetch():
                fetch_idx(j + AH_F, qf)

            def wait_scatter_prev():
                pltpu.make_async_copy(
                    rows[rbg], acc.at[idx[qp].at[1]], ssem[rbg]).wait()

            def fire_gather_next():
                wait_idx(j + AH_G, qg)
                pltpu.async_copy(
                    h_hbm.at[pl.ds(((j + AH_G) * CH) % 9920, CH)],
                    rows[rbg], gsem[rbg])

            if static:
                if j + AH_F < NCHUNK:
                    fire_fetch()
                if j + AH_G < NCHUNK:
                    fire_gather_next()
            else:
                if (NGRP - 1) * GRP + b + AH_F >= NCHUNK:
                    @pl.when(j + AH_F < NCHUNK)
                    def _():
                        fire_fetch()
                else:
                    fire_fetch()
                fire_gather_next()
            pltpu.make_async_copy(
                h_hbm.at[pl.ds((j * CH) % 9920, CH)], rows[rb], gsem[rb]).wait()

        def group(g, carry):
            for b in range(GRP):
                emit_steady(g, b)
            return carry

        lax.fori_loop(0, NGRP, group, 0)

        for j in range(EPI0, NCHUNK):
            emit_steady(j // GRP, j % GRP)

        plsc.subcore_barrier()

        for t in range(NSTRIPE):
            r = row0 + t * ZROWS
            pltpu.async_copy(acc.at[pl.ds(r, ZROWS)],
                             out_hbm.at[c, pl.ds(r, ZROWS)], gsem[0])
        for t in range(NSTRIPE):
            r = row0 + t * ZROWS
            pltpu.make_async_copy(acc.at[pl.ds(r, ZROWS)],
                                  out_hbm.at[c, pl.ds(r, ZROWS)], gsem[0]).wait()

    return k(h, edge_index)


def _combine(p, h, Wr, Wroot, b2d, relu):
    BR = 1000

    def body(p_ref, h_ref, wr_ref, wroot_ref, b_ref, o_ref):
        agg = p_ref[0] + p_ref[1]
        out = jnp.dot(agg, wr_ref[...], preferred_element_type=jnp.float32)
        out = out + jnp.dot(h_ref[...], wroot_ref[...],
                            preferred_element_type=jnp.float32)
        out = out + b_ref[...]
        if relu:
            out = jnp.maximum(out, 0.0)
        o_ref[...] = out

    return pl.pallas_call(
        body,
        grid=(N // BR,),
        in_specs=[
            pl.BlockSpec((NC, BR, D), lambda i: (0, i, 0)),
            pl.BlockSpec((BR, D), lambda i: (i, 0)),
            pl.BlockSpec((D, D), lambda i: (0, 0)),
            pl.BlockSpec((D, D), lambda i: (0, 0)),
            pl.BlockSpec((1, D), lambda i: (0, 0)),
        ],
        out_specs=pl.BlockSpec((BR, D), lambda i: (i, 0)),
        out_shape=jax.ShapeDtypeStruct((N, D), jnp.float32),
    )(p, h, Wr, Wroot, b2d)


def kernel(x, edge_index, Wr0, Wroot0, b0, Wr1, Wroot1, b1, Wr2, Wroot2, b2):
    ei = edge_index.astype(jnp.int32).reshape(2 * E)
    h = x
    for Wr, Wroot, b, relu in ((Wr0, Wroot0, b0, True),
                               (Wr1, Wroot1, b1, True),
                               (Wr2, Wroot2, b2, False)):
        p = _seg_sum_partials(h, ei)
        h = _combine(p, h, Wr, Wroot, b.reshape(1, D), relu)
    return h


# speedup vs baseline: 1.1282x; 1.1282x over previous
import functools

import jax
import jax.numpy as jnp
from jax import lax
from jax.experimental import pallas as pl
from jax.experimental.pallas import tpu as pltpu
from jax.experimental.pallas import tpu_sc as plsc

N = 10000
E = 320000
D = 128

NC = 2
NS = 16
NPAD = 10240
CH = 80
EDGES_PER_TILE = E // (NC * NS)
NCHUNK = EDGES_PER_TILE // CH
ROWS_PER_TILE = NPAD // NS
ZROWS = 128
NSTRIPE = ROWS_PER_TILE // ZROWS

NBUF = 4
NIDX = 8
AH_G = 3
AH_F = 6
GRP = 8
NGRP = (NCHUNK - GRP + AH_G) // GRP
EPI0 = NGRP * GRP


def _seg_sum_partials(h2, edge_index):
    mesh = plsc.VectorSubcoreMesh(core_axis_name="c", subcore_axis_name="s")

    @functools.partial(
        pl.kernel,
        mesh=mesh,
        out_type=jax.ShapeDtypeStruct((NC, NPAD, D), jnp.float32),
        scratch_types=(
            [pltpu.VMEM((CH, D), jnp.float32) for _ in range(NBUF)]
            + [pltpu.VMEM((2, CH), jnp.int32) for _ in range(NIDX)]
            + [pltpu.VMEM_SHARED((NPAD, D), jnp.float32)]
            + [pltpu.SemaphoreType.DMA] * (2 * NBUF + NIDX)
        ),
    )
    def k(h2_hbm, ei_hbm, out_hbm, *refs):
        rows = refs[0:NBUF]
        idx = refs[NBUF:NBUF + NIDX]
        acc = refs[NBUF + NIDX]
        sems = refs[NBUF + NIDX + 1:]
        gsem = sems[0:NBUF]
        ssem = sems[NBUF:2 * NBUF]
        isem = sems[2 * NBUF:]

        c = lax.axis_index("c")
        s = lax.axis_index("s")
        base = c * (E // NC) + s * EDGES_PER_TILE
        h_hbm = h2_hbm.at[c]

        def fetch_idx(j, q):
            off = base + j * CH
            pltpu.async_copy(ei_hbm.at[pl.ds(off, CH)], idx[q].at[0], isem[q])
            pltpu.async_copy(
                ei_hbm.at[pl.ds(E + off, CH)], idx[q].at[1], isem[q])

        def wait_idx(j, q):
            off = base + j * CH
            pltpu.make_async_copy(
                ei_hbm.at[pl.ds(off, CH)], idx[q].at[0], isem[q]).wait()
            pltpu.make_async_copy(
                ei_hbm.at[pl.ds(E + off, CH)], idx[q].at[1], isem[q]).wait()

        for j in range(AH_F):
            fetch_idx(j, j)

        zb = NBUF - 1

        def zfill(t, carry):
            i = t // (D // 16)
            j = t % (D // 16)
            rows[zb][i, pl.ds(j * 16, 16)] = jnp.zeros((16,), jnp.float32)
            return carry

        lax.fori_loop(0, CH * (D // 16), zfill, 0)

        row0 = s * ROWS_PER_TILE
        for t in range(ROWS_PER_TILE // CH):
            pltpu.async_copy(rows[zb], acc.at[pl.ds(row0 + t * CH, CH)],
                             ssem[zb])
        for j in range(AH_G):
            wait_idx(j, j)
            pltpu.async_copy(h_hbm.at[idx[j].at[0]], rows[j], gsem[j])
        for t in range(ROWS_PER_TILE // CH):
            pltpu.make_async_copy(
                rows[zb], acc.at[pl.ds(row0 + t * CH, CH)], ssem[zb]).wait()
        plsc.subcore_barrier()

        def emit_steady(g, b):
            j = g * GRP + b
            static = isinstance(j, int)
            jprev = NBUF - AH_G
            rb = b % NBUF
            rbg = (b + AH_G) % NBUF
            q = b % NIDX
            qg = (b + AH_G) % NIDX
            qf = (b + AH_F) % NIDX
            qp = (b + NIDX - jprev) % NIDX

            def fire_fetch():
                fetch_idx(j + AH_F, qf)

            def wait_scatter_prev():
                pltpu.make_async_copy(
                    rows[rbg], acc.at[idx[qp].at[1]], ssem[rbg]).wait()

            def fire_gather_next():
                wait_idx(j + AH_G, qg)
                pltpu.async_copy(h_hbm.at[idx[qg].at[0]], rows[rbg], gsem[rbg])

            if static:
                if j + AH_F < NCHUNK:
                    fire_fetch()
                wait_scatter_prev()
                if j + AH_G < NCHUNK:
                    fire_gather_next()
            else:
                if (NGRP - 1) * GRP + b + AH_F >= NCHUNK:
                    @pl.when(j + AH_F < NCHUNK)
                    def _():
                        fire_fetch()
                else:
                    fire_fetch()
                if b < jprev:
                    @pl.when(j >= jprev)
                    def _():
                        wait_scatter_prev()
                else:
                    wait_scatter_prev()
                fire_gather_next()
            pltpu.make_async_copy(
                h_hbm.at[idx[q].at[0]], rows[rb], gsem[rb]).wait()
            pltpu.async_copy(rows[rb], acc.at[idx[q].at[1]], ssem[rb], add=True)

        def group(g, carry):
            for b in range(GRP):
                emit_steady(g, b)
            return carry

        lax.fori_loop(0, NGRP, group, 0)

        for j in range(EPI0, NCHUNK):
            emit_steady(j // GRP, j % GRP)

        for j in range(NCHUNK - (NBUF - AH_G), NCHUNK):
            pltpu.make_async_copy(
                rows[(j % GRP) % NBUF], acc.at[idx[j % NIDX].at[1]],
                ssem[(j % GRP) % NBUF]).wait()
        plsc.subcore_barrier()

        for t in range(NSTRIPE):
            r = row0 + t * ZROWS
            pltpu.async_copy(acc.at[pl.ds(r, ZROWS)],
                             out_hbm.at[c, pl.ds(r, ZROWS)], gsem[0])
        for t in range(NSTRIPE):
            r = row0 + t * ZROWS
            pltpu.make_async_copy(acc.at[pl.ds(r, ZROWS)],
                                  out_hbm.at[c, pl.ds(r, ZROWS)], gsem[0]).wait()

    return k(h2, edge_index)


def _combine(p, h2, Wr, Wroot, b2d, relu, dup):
    BR = 1000

    def body(p_ref, h_ref, wr_ref, wroot_ref, b_ref, o_ref):
        agg = p_ref[0] + p_ref[1]
        out = jnp.dot(agg, wr_ref[...], preferred_element_type=jnp.float32)
        out = out + jnp.dot(h_ref[0], wroot_ref[...],
                            preferred_element_type=jnp.float32)
        out = out + b_ref[...]
        if relu:
            out = jnp.maximum(out, 0.0)
        if dup:
            o_ref[0] = out
            o_ref[1] = out
        else:
            o_ref[...] = out

    if dup:
        out_spec = pl.BlockSpec((NC, BR, D), lambda i: (0, i, 0))
        out_shape = jax.ShapeDtypeStruct((NC, N, D), jnp.float32)
    else:
        out_spec = pl.BlockSpec((BR, D), lambda i: (i, 0))
        out_shape = jax.ShapeDtypeStruct((N, D), jnp.float32)

    return pl.pallas_call(
        body,
        grid=(N // BR,),
        in_specs=[
            pl.BlockSpec((NC, BR, D), lambda i: (0, i, 0)),
            pl.BlockSpec((1, BR, D), lambda i: (0, i, 0)),
            pl.BlockSpec((D, D), lambda i: (0, 0)),
            pl.BlockSpec((D, D), lambda i: (0, 0)),
            pl.BlockSpec((1, D), lambda i: (0, 0)),
        ],
        out_specs=out_spec,
        out_shape=out_shape,
    )(p, h2, Wr, Wroot, b2d)


def kernel(x, edge_index, Wr0, Wroot0, b0, Wr1, Wroot1, b1, Wr2, Wroot2, b2):
    ei = edge_index.astype(jnp.int32).reshape(2 * E)
    h2 = jnp.stack([x, x])
    for Wr, Wroot, b, relu, dup in ((Wr0, Wroot0, b0, True, True),
                                    (Wr1, Wroot1, b1, True, True),
                                    (Wr2, Wroot2, b2, False, False)):
        p = _seg_sum_partials(h2, ei)
        h2 = _combine(p, h2, Wr, Wroot, b.reshape(1, D), relu, dup)
    return h2


# speedup vs baseline: 1.1676x; 1.0348x over previous
import functools

import jax
import jax.numpy as jnp
from jax import lax
from jax.experimental import pallas as pl
from jax.experimental.pallas import tpu as pltpu
from jax.experimental.pallas import tpu_sc as plsc

N = 10000
E = 320000
D = 128

NC = 2
NS = 16
NPAD = 10240
CH = 80
EDGES_PER_TILE = E // (NC * NS)
NCHUNK = EDGES_PER_TILE // CH
ROWS_PER_TILE = NPAD // NS
ZROWS = 128
NSTRIPE = ROWS_PER_TILE // ZROWS

NBUF = 4
NIDX = 8
GRP = 8
NGRP = 15


def _seg_sum_partials(h, edge_index):
    mesh = plsc.VectorSubcoreMesh(core_axis_name="c", subcore_axis_name="s")

    @functools.partial(
        pl.kernel,
        mesh=mesh,
        out_type=jax.ShapeDtypeStruct((NC, NPAD, D), jnp.float32),
        scratch_types=(
            [pltpu.VMEM((CH, D), jnp.float32) for _ in range(NBUF)]
            + [pltpu.VMEM((2, CH), jnp.int32) for _ in range(NIDX)]
            + [pltpu.VMEM_SHARED((NPAD, D), jnp.float32)]
            + [pltpu.SemaphoreType.DMA] * (2 * NBUF + NIDX)
        ),
    )
    def k(h_hbm, ei_hbm, out_hbm, *refs):
        rows = refs[0:NBUF]
        idx = refs[NBUF:NBUF + NIDX]
        acc = refs[NBUF + NIDX]
        sems = refs[NBUF + NIDX + 1:]
        gsem = sems[0:NBUF]
        ssem = sems[NBUF:2 * NBUF]
        isem = sems[2 * NBUF:]

        c = lax.axis_index("c")
        s = lax.axis_index("s")
        base = c * (E // NC) + s * EDGES_PER_TILE

        def fetch_idx(j, q):
            off = base + j * CH
            pltpu.async_copy(ei_hbm.at[pl.ds(off, CH)], idx[q].at[0], isem[q])
            pltpu.async_copy(
                ei_hbm.at[pl.ds(E + off, CH)], idx[q].at[1], isem[q])

        def wait_idx(j, q):
            off = base + j * CH
            pltpu.make_async_copy(
                ei_hbm.at[pl.ds(off, CH)], idx[q].at[0], isem[q]).wait()
            pltpu.make_async_copy(
                ei_hbm.at[pl.ds(E + off, CH)], idx[q].at[1], isem[q]).wait()

        def zfill(t, carry):
            i = t // (D // 16)
            j = t % (D // 16)
            rows[0][i, pl.ds(j * 16, 16)] = jnp.zeros((16,), jnp.float32)
            return carry

        lax.fori_loop(0, CH * (D // 16), zfill, 0)

        row0 = s * ROWS_PER_TILE
        for t in range(ROWS_PER_TILE // CH):
            pltpu.async_copy(rows[0], acc.at[pl.ds(row0 + t * CH, CH)], gsem[0])
        for t in range(ROWS_PER_TILE // CH):
            pltpu.make_async_copy(
                rows[0], acc.at[pl.ds(row0 + t * CH, CH)], gsem[0]).wait()
        plsc.subcore_barrier()

        for j in range(6):
            fetch_idx(j, j)
        for j in range(3):
            wait_idx(j, j)
            pltpu.async_copy(h_hbm.at[idx[j].at[0]], rows[j], gsem[j])

        def emit_steady(g, b):
            j = g * GRP + b
            static = isinstance(j, int)
            rb = b % NBUF
            rb3 = (b + 3) % NBUF
            q = b % NIDX
            q3 = (b + 3) % NIDX
            q6 = (b + 6) % NIDX
            qp = (b + NIDX - 1) % NIDX

            def fire_fetch():
                fetch_idx(j + 6, q6)

            def wait_scatter_prev():
                pltpu.make_async_copy(
                    rows[rb3], acc.at[idx[qp].at[1]], ssem[rb3]).wait()

            def fire_gather_next():
                wait_idx(j + 3, q3)
                pltpu.async_copy(h_hbm.at[idx[q3].at[0]], rows[rb3], gsem[rb3])

            if static:
                if j + 6 < NCHUNK:
                    fire_fetch()
                wait_scatter_prev()
                if j + 3 < NCHUNK:
                    fire_gather_next()
            else:
                if b == 7:
                    @pl.when(j + 6 < NCHUNK)
                    def _():
                        fire_fetch()
                else:
                    fire_fetch()
                if b == 0:
                    @pl.when(j >= 1)
                    def _():
                        wait_scatter_prev()
                else:
                    wait_scatter_prev()
                fire_gather_next()
            pltpu.make_async_copy(
                h_hbm.at[idx[q].at[0]], rows[rb], gsem[rb]).wait()
            pltpu.async_copy(rows[rb], acc.at[idx[q].at[1]], ssem[rb], add=True)

        def group(g, carry):
            for b in range(GRP):
                emit_steady(g, b)
            return carry

        lax.fori_loop(0, NGRP, group, 0)

        for j in range(NGRP * GRP, NCHUNK):
            emit_steady(j // GRP, j % GRP)

        jl = NCHUNK - 1
        pltpu.make_async_copy(
            rows[(jl % GRP) % NBUF], acc.at[idx[jl % NIDX].at[1]],
            ssem[(jl % GRP) % NBUF]).wait()
        plsc.subcore_barrier()

        for t in range(NSTRIPE):
            r = row0 + t * ZROWS
            pltpu.async_copy(acc.at[pl.ds(r, ZROWS)],
                             out_hbm.at[c, pl.ds(r, ZROWS)], gsem[0])
        for t in range(NSTRIPE):
            r = row0 + t * ZROWS
            pltpu.make_async_copy(acc.at[pl.ds(r, ZROWS)],
                                  out_hbm.at[c, pl.ds(r, ZROWS)], gsem[0]).wait()

    return k(h, edge_index)


def _combine(p, h, Wr, Wroot, b2d, relu):
    BR = 2000

    def body(p_ref, h_ref, wr_ref, wroot_ref, b_ref, o_ref):
        agg = p_ref[0] + p_ref[1]
        out = jnp.dot(agg, wr_ref[...], preferred_element_type=jnp.float32)
        out = out + jnp.dot(h_ref[...], wroot_ref[...],
                            preferred_element_type=jnp.float32)
        out = out + b_ref[...]
        if relu:
            out = jnp.maximum(out, 0.0)
        o_ref[...] = out

    return pl.pallas_call(
        body,
        grid=(N // BR,),
        in_specs=[
            pl.BlockSpec((NC, BR, D), lambda i: (0, i, 0)),
            pl.BlockSpec((BR, D), lambda i: (i, 0)),
            pl.BlockSpec((D, D), lambda i: (0, 0)),
            pl.BlockSpec((D, D), lambda i: (0, 0)),
            pl.BlockSpec((1, D), lambda i: (0, 0)),
        ],
        out_specs=pl.BlockSpec((BR, D), lambda i: (i, 0)),
        out_shape=jax.ShapeDtypeStruct((N, D), jnp.float32),
    )(p, h, Wr, Wroot, b2d)


def kernel(x, edge_index, Wr0, Wroot0, b0, Wr1, Wroot1, b1, Wr2, Wroot2, b2):
    ei = edge_index.astype(jnp.int32).reshape(2 * E)
    h = x
    for Wr, Wroot, b, relu in ((Wr0, Wroot0, b0, True),
                               (Wr1, Wroot1, b1, True),
                               (Wr2, Wroot2, b2, False)):
        p = _seg_sum_partials(h, ei)
        h = _combine(p, h, Wr, Wroot, b.reshape(1, D), relu)
    return h
